# baseline (speedup 1.0000x reference)
import functools

import jax
import jax.numpy as jnp
from jax import lax
from jax.experimental import pallas as pl
from jax.experimental.pallas import tpu as pltpu
from jax.experimental.pallas import tpu_sc as plsc


_TINY = float(jnp.finfo(jnp.float32).tiny)


def _neg_gumbel_chunk(k1, k2, base_row, coli, v):
    bm, cw = coli.shape
    rowi = lax.broadcasted_iota(jnp.int32, (bm, cw), 0)
    p = ((base_row + rowi) * v + coli).astype(jnp.uint32)

    def rotl(x, d):
        return (x << jnp.uint32(d)) | (x >> jnp.uint32(32 - d))

    ks0, ks1 = k1, k2
    ks2 = k1 ^ k2 ^ jnp.uint32(0x1BD11BDA)
    x0 = jnp.full((bm, cw), ks0, dtype=jnp.uint32)
    x1 = p + ks1

    def rounds(x0, x1, rots):
        for r in rots:
            x0 = x0 + x1
            x1 = rotl(x1, r)
            x1 = x0 ^ x1
        return x0, x1

    ra = (13, 15, 26, 6)
    rb = (17, 29, 16, 24)
    x0, x1 = rounds(x0, x1, ra)
    x0 = x0 + ks1
    x1 = x1 + (ks2 + jnp.uint32(1))
    x0, x1 = rounds(x0, x1, rb)
    x0 = x0 + ks2
    x1 = x1 + (ks0 + jnp.uint32(2))
    x0, x1 = rounds(x0, x1, ra)
    x0 = x0 + ks0
    x1 = x1 + (ks1 + jnp.uint32(3))
    x0, x1 = rounds(x0, x1, rb)
    x0 = x0 + ks1
    x1 = x1 + (ks2 + jnp.uint32(4))
    x0, x1 = rounds(x0, x1, ra)
    x0 = x0 + ks2
    x1 = x1 + (ks0 + jnp.uint32(5))
    bits = x0 ^ x1

    float_bits = (bits >> jnp.uint32(9)) | jnp.uint32(0x3F800000)
    floats = lax.bitcast_convert_type(float_bits, jnp.float32) - jnp.float32(1.0)
    u = floats + jnp.float32(_TINY)
    return jnp.log(-jnp.log(u))


def _mlp_sample_body(key_ref, x_ref, w1_ref, b1_ref, w2_ref, b2_ref, w3_ref,
                     b3_ref, lat_ref, lp_ref):
    bm = x_ref.shape[0]
    v = w3_ref.shape[1]
    k1 = key_ref[0].astype(jnp.uint32)
    k2 = key_ref[1].astype(jnp.uint32)
    base_row = pl.program_id(0) * bm

    h = jnp.maximum(
        jnp.dot(x_ref[...], w1_ref[...], preferred_element_type=jnp.float32)
        + b1_ref[...], 0.0)
    h2 = jnp.maximum(
        jnp.dot(h, w2_ref[...], preferred_element_type=jnp.float32)
        + b2_ref[...], 0.0)
    logits = (jnp.dot(h2, w3_ref[...], preferred_element_type=jnp.float32)
              + b3_ref[...])

    coli = lax.broadcasted_iota(jnp.int32, (bm, v), 1)
    neg_g = _neg_gumbel_chunk(k1, k2, base_row, coli, v)
    a = logits - neg_g
    amax = jnp.max(a, axis=1, keepdims=True)
    col = lax.broadcasted_iota(jnp.int32, (bm, v), 1)
    idx = jnp.min(jnp.where(a == amax, col, v), axis=1)
    mx = jnp.max(logits, axis=1, keepdims=True)
    lse = jnp.log(jnp.sum(jnp.exp(logits - mx), axis=1)) + mx[:, 0]
    sel = jnp.sum(jnp.where(col == idx[:, None], logits, 0.0), axis=1)
    lat_ref[0, 0, :] = idx
    lp_ref[0, 0, :] = sel - lse


def _mlp_sample(x, W1, b1, W2, b2, W3, b3, key_bits, bm):
    B, F = x.shape
    H = W1.shape[1]
    V = W3.shape[1]
    nb = B // bm
    lat3, lp3 = pl.pallas_call(
        _mlp_sample_body,
        grid=(nb,),
        in_specs=[
            pl.BlockSpec(memory_space=pltpu.SMEM),
            pl.BlockSpec((bm, F), lambda i: (i, 0)),
            pl.BlockSpec((F, H), lambda i: (0, 0)),
            pl.BlockSpec((1, H), lambda i: (0, 0)),
            pl.BlockSpec((H, H), lambda i: (0, 0)),
            pl.BlockSpec((1, H), lambda i: (0, 0)),
            pl.BlockSpec((H, V), lambda i: (0, 0)),
            pl.BlockSpec((1, V), lambda i: (0, 0)),
        ],
        out_specs=[
            pl.BlockSpec((1, 1, bm), lambda i: (i, 0, 0)),
            pl.BlockSpec((1, 1, bm), lambda i: (i, 0, 0)),
        ],
        out_shape=[
            jax.ShapeDtypeStruct((nb, 1, bm), jnp.int32),
            jax.ShapeDtypeStruct((nb, 1, bm), jnp.float32),
        ],
        compiler_params=pltpu.CompilerParams(
            dimension_semantics=("parallel",)),
    )(key_bits, x, W1, b1.reshape(1, H), W2, b2.reshape(1, H), W3,
      b3.reshape(1, V))
    return lat3.reshape(B), lp3.reshape(B)


def _sc_gather(mu, log_var, latent):
    _, F = mu.shape
    B = latent.shape[0]
    info = plsc.get_sparse_core_info()
    nc = info.num_cores
    nw = nc * info.num_subcores
    bpw = B // nw
    mesh = plsc.VectorSubcoreMesh(core_axis_name="c", subcore_axis_name="s")

    nch = 4
    ch = bpw // nch

    @functools.partial(
        pl.kernel, mesh=mesh,
        out_type=(jax.ShapeDtypeStruct((B, F), jnp.float32),
                  jax.ShapeDtypeStruct((B, F), jnp.float32)),
        scratch_types=[
            pltpu.VMEM((bpw,), jnp.int32),
            pltpu.VMEM((ch, F), jnp.float32),
            pltpu.VMEM((ch, F), jnp.float32),
            pltpu.VMEM((ch, F), jnp.float32),
            pltpu.VMEM((ch, F), jnp.float32),
            pltpu.SemaphoreType.DMA,
            pltpu.SemaphoreType.DMA,
            pltpu.SemaphoreType.DMA,
            pltpu.SemaphoreType.DMA,
        ],
    )
    def gather_k(mu_hbm, lv_hbm, idx_hbm, mu_out, lv_out,
                 idx_v, mA, lA, mB, lB, sgA, sgB, swA, swB):
        wid = lax.axis_index("s") * nc + lax.axis_index("c")
        base = wid * bpw
        pltpu.sync_copy(idx_hbm.at[pl.ds(base, bpw)], idx_v)
        bufs = ((mA, lA, sgA, swA), (mB, lB, sgB, swB))
        pend_g = [None, None]
        pend_w = [None, None]
        for c in range(nch):
            b = c % 2
            m, l, sg, sw = bufs[b]
            if pend_w[b] is not None:
                for w in pend_w[b]:
                    w.wait()
                pend_w[b] = None
            idx_c = idx_v.at[pl.ds(c * ch, ch)]
            pend_g[b] = (c, pltpu.async_copy(mu_hbm.at[idx_c], m, sg),
                         pltpu.async_copy(lv_hbm.at[idx_c], l, sg))
            pb = (c + 1) % 2
            if pend_g[pb] is not None:
                pc, gm, gl = pend_g[pb]
                mp, lp, _, swp = bufs[pb]
                gm.wait()
                gl.wait()
                pend_w[pb] = (
                    pltpu.async_copy(mp, mu_out.at[pl.ds(base + pc * ch, ch)],
                                     swp),
                    pltpu.async_copy(lp, lv_out.at[pl.ds(base + pc * ch, ch)],
                                     swp),
                )
                pend_g[pb] = None
        lb = (nch - 1) % 2
        pc, gm, gl = pend_g[lb]
        mp, lp, _, swp = bufs[lb]
        gm.wait()
        gl.wait()
        pend_w[lb] = (
            pltpu.async_copy(mp, mu_out.at[pl.ds(base + pc * ch, ch)], swp),
            pltpu.async_copy(lp, lv_out.at[pl.ds(base + pc * ch, ch)], swp),
        )
        for b in range(2):
            if pend_w[b] is not None:
                for w in pend_w[b]:
                    w.wait()

    return gather_k(mu, log_var, latent)


def kernel(x, W1, b1, W2, b2, W3, b3, mu, log_var, sample_key):
    key_bits = lax.bitcast_convert_type(
        jax.random.key_data(sample_key).astype(jnp.uint32), jnp.int32)
    latent, latent_log_p = _mlp_sample(x, W1, b1, W2, b2, W3, b3, key_bits,
                                       bm=1024)
    mu_out, lv_out = _sc_gather(mu, log_var, latent)
    return (mu_out, lv_out, latent_log_p)


# speedup vs baseline: 1.0005x; 1.0005x over previous
import functools

import jax
import jax.numpy as jnp
from jax import lax
from jax.experimental import pallas as pl
from jax.experimental.pallas import tpu as pltpu
from jax.experimental.pallas import tpu_sc as plsc


_TINY = float(jnp.finfo(jnp.float32).tiny)


def _neg_gumbel_chunk(k1, k2, base_row, coli, v):
    bm, cw = coli.shape
    rowi = lax.broadcasted_iota(jnp.int32, (bm, cw), 0)
    p = ((base_row + rowi) * v + coli).astype(jnp.uint32)

    def rotl(x, d):
        return (x << jnp.uint32(d)) | (x >> jnp.uint32(32 - d))

    ks0, ks1 = k1, k2
    ks2 = k1 ^ k2 ^ jnp.uint32(0x1BD11BDA)
    x0 = jnp.full((bm, cw), ks0, dtype=jnp.uint32)
    x1 = p + ks1

    def rounds(x0, x1, rots):
        for r in rots:
            x0 = x0 + x1
            x1 = rotl(x1, r)
            x1 = x0 ^ x1
        return x0, x1

    ra = (13, 15, 26, 6)
    rb = (17, 29, 16, 24)
    x0, x1 = rounds(x0, x1, ra)
    x0 = x0 + ks1
    x1 = x1 + (ks2 + jnp.uint32(1))
    x0, x1 = rounds(x0, x1, rb)
    x0 = x0 + ks2
    x1 = x1 + (ks0 + jnp.uint32(2))
    x0, x1 = rounds(x0, x1, ra)
    x0 = x0 + ks0
    x1 = x1 + (ks1 + jnp.uint32(3))
    x0, x1 = rounds(x0, x1, rb)
    x0 = x0 + ks1
    x1 = x1 + (ks2 + jnp.uint32(4))
    x0, x1 = rounds(x0, x1, ra)
    x0 = x0 + ks2
    x1 = x1 + (ks0 + jnp.uint32(5))
    bits = x0 ^ x1

    float_bits = (bits >> jnp.uint32(9)) | jnp.uint32(0x3F800000)
    floats = lax.bitcast_convert_type(float_bits, jnp.float32) - jnp.float32(1.0)
    u = floats + jnp.float32(_TINY)
    return jnp.log(-jnp.log(u))


def _mlp_sample_body(key_ref, x_ref, w1_ref, b1_ref, w2_ref, b2_ref, w3_ref,
                     b3_ref, lat_ref, lp_ref):
    bm = x_ref.shape[0]
    v = w3_ref.shape[1]
    k1 = key_ref[0].astype(jnp.uint32)
    k2 = key_ref[1].astype(jnp.uint32)
    base_row = pl.program_id(0) * bm

    h = jnp.maximum(
        jnp.dot(x_ref[...], w1_ref[...], preferred_element_type=jnp.float32)
        + b1_ref[...], 0.0)
    h2 = jnp.maximum(
        jnp.dot(h, w2_ref[...], preferred_element_type=jnp.float32)
        + b2_ref[...], 0.0)
    logits = (jnp.dot(h2, w3_ref[...], preferred_element_type=jnp.float32)
              + b3_ref[...])

    col = lax.broadcasted_iota(jnp.int32, (bm, v), 1)
    neg_g = _neg_gumbel_chunk(k1, k2, base_row, col, v)
    a = logits - neg_g
    amax = jnp.max(a, axis=1, keepdims=True)
    idx = jnp.min(jnp.where(a == amax, col, v), axis=1)
    mx = jnp.max(logits, axis=1, keepdims=True)
    lse = jnp.log(jnp.sum(jnp.exp(logits - mx), axis=1)) + mx[:, 0]
    sel = jnp.sum(jnp.where(col == idx[:, None], logits, 0.0), axis=1)
    lat_ref[0, 0, :] = idx
    lp_ref[0, 0, :] = sel - lse


def _mlp_sample(x, W1, b1, W2, b2, W3, b3, key_bits, bm):
    B, F = x.shape
    H = W1.shape[1]
    V = W3.shape[1]
    nb = B // bm
    lat3, lp3 = pl.pallas_call(
        _mlp_sample_body,
        grid=(nb,),
        in_specs=[
            pl.BlockSpec(memory_space=pltpu.SMEM),
            pl.BlockSpec((bm, F), lambda i: (i, 0)),
            pl.BlockSpec((F, H), lambda i: (0, 0)),
            pl.BlockSpec((1, H), lambda i: (0, 0)),
            pl.BlockSpec((H, H), lambda i: (0, 0)),
            pl.BlockSpec((1, H), lambda i: (0, 0)),
            pl.BlockSpec((H, V), lambda i: (0, 0)),
            pl.BlockSpec((1, V), lambda i: (0, 0)),
        ],
        out_specs=[
            pl.BlockSpec((1, 1, bm), lambda i: (i, 0, 0)),
            pl.BlockSpec((1, 1, bm), lambda i: (i, 0, 0)),
        ],
        out_shape=[
            jax.ShapeDtypeStruct((nb, 1, bm), jnp.int32),
            jax.ShapeDtypeStruct((nb, 1, bm), jnp.float32),
        ],
        compiler_params=pltpu.CompilerParams(
            dimension_semantics=("parallel",)),
    )(key_bits, x, W1, b1.reshape(1, H), W2, b2.reshape(1, H), W3,
      b3.reshape(1, V))
    return lat3.reshape(B), lp3.reshape(B)


def _sc_gather(mu, log_var, latent):
    _, F = mu.shape
    B = latent.shape[0]
    info = plsc.get_sparse_core_info()
    nc = info.num_cores
    nw = nc * info.num_subcores
    bpw = B // nw
    mesh = plsc.VectorSubcoreMesh(core_axis_name="c", subcore_axis_name="s")

    nch = 4
    ch = bpw // nch

    @functools.partial(
        pl.kernel, mesh=mesh,
        out_type=(jax.ShapeDtypeStruct((B, F), jnp.float32),
                  jax.ShapeDtypeStruct((B, F), jnp.float32)),
        scratch_types=[
            pltpu.VMEM((bpw,), jnp.int32),
            pltpu.VMEM((ch, F), jnp.float32),
            pltpu.VMEM((ch, F), jnp.float32),
            pltpu.VMEM((ch, F), jnp.float32),
            pltpu.VMEM((ch, F), jnp.float32),
            pltpu.SemaphoreType.DMA,
            pltpu.SemaphoreType.DMA,
            pltpu.SemaphoreType.DMA,
            pltpu.SemaphoreType.DMA,
        ],
    )
    def gather_k(mu_hbm, lv_hbm, idx_hbm, mu_out, lv_out,
                 idx_v, mA, lA, mB, lB, sgA, sgB, swA, swB):
        wid = lax.axis_index("s") * nc + lax.axis_index("c")
        base = wid * bpw
        pltpu.sync_copy(idx_hbm.at[pl.ds(base, bpw)], idx_v)
        bufs = ((mA, lA, sgA, swA), (mB, lB, sgB, swB))
        pend_g = [None, None]
        pend_w = [None, None]
        for c in range(nch):
            b = c % 2
            m, l, sg, sw = bufs[b]
            if pend_w[b] is not None:
                for w in pend_w[b]:
                    w.wait()
                pend_w[b] = None
            idx_c = idx_v.at[pl.ds(c * ch, ch)]
            pend_g[b] = (c, pltpu.async_copy(mu_hbm.at[idx_c], m, sg),
                         pltpu.async_copy(lv_hbm.at[idx_c], l, sg))
            pb = (c + 1) % 2
            if pend_g[pb] is not None:
                pc, gm, gl = pend_g[pb]
                mp, lp, _, swp = bufs[pb]
                gm.wait()
                gl.wait()
                pend_w[pb] = (
                    pltpu.async_copy(mp, mu_out.at[pl.ds(base + pc * ch, ch)],
                                     swp),
                    pltpu.async_copy(lp, lv_out.at[pl.ds(base + pc * ch, ch)],
                                     swp),
                )
                pend_g[pb] = None
        lb = (nch - 1) % 2
        pc, gm, gl = pend_g[lb]
        mp, lp, _, swp = bufs[lb]
        gm.wait()
        gl.wait()
        pend_w[lb] = (
            pltpu.async_copy(mp, mu_out.at[pl.ds(base + pc * ch, ch)], swp),
            pltpu.async_copy(lp, lv_out.at[pl.ds(base + pc * ch, ch)], swp),
        )
        for b in range(2):
            if pend_w[b] is not None:
                for w in pend_w[b]:
                    w.wait()

    return gather_k(mu, log_var, latent)


def kernel(x, W1, b1, W2, b2, W3, b3, mu, log_var, sample_key):
    key_bits = lax.bitcast_convert_type(
        jax.random.key_data(sample_key).astype(jnp.uint32), jnp.int32)
    latent, latent_log_p = _mlp_sample(x, W1, b1, W2, b2, W3, b3, key_bits,
                                       bm=1024)
    mu_out, lv_out = _sc_gather(mu, log_var, latent)
    return (mu_out, lv_out, latent_log_p)
